# baseline (speedup 1.0000x reference)
import functools

import jax
import jax.numpy as jnp
from jax import lax
from jax.experimental import pallas as pl
from jax.experimental.pallas import tpu as pltpu
from jax.experimental.pallas import tpu_sc as plsc


@functools.lru_cache(maxsize=None)
def _build_gather(BATCH: int, SP: int, V: int, DB: int):
    info = plsc.get_sparse_core_info()
    nc, ns = info.num_cores, info.num_subcores
    nw = nc * ns
    assert BATCH % nw == 0
    bpw = BATCH // nw
    assert bpw % 2 == 0 and bpw >= 4
    DO = DB - 1

    mesh = plsc.VectorSubcoreMesh(core_axis_name="c", subcore_axis_name="s")

    @functools.partial(
        pl.kernel,
        mesh=mesh,
        compiler_params=pltpu.CompilerParams(use_tc_tiling_on_sc=False),
        out_type=jax.ShapeDtypeStruct((BATCH, SP, DO, 128), jnp.float32),
        scratch_types=[
            pltpu.VMEM((bpw, SP), jnp.int32),
            pltpu.VMEM((2, SP, DB, 128), jnp.float32),
            pltpu.SemaphoreType.DMA,
            pltpu.SemaphoreType.DMA,
            pltpu.SemaphoreType.DMA,
            pltpu.SemaphoreType.DMA,
        ],
    )
    def k(idx_hbm, table_hbm, out_hbm, idx_v, rows_v, g0, g1, s0, s1):
        gsem = (g0, g1)
        ssem = (s0, s1)
        wid = lax.axis_index("s") * nc + lax.axis_index("c")
        base = wid * bpw
        pltpu.sync_copy(idx_hbm.at[pl.ds(base, bpw)], idx_v)

        def start_gather(i, b):
            pltpu.async_copy(table_hbm.at[idx_v.at[i]], rows_v.at[b], gsem[b])

        def wait_gather(b):
            pltpu.make_async_copy(
                table_hbm.at[pl.ds(0, SP)], rows_v.at[b], gsem[b]
            ).wait()

        def start_store(i, b):
            pltpu.async_copy(
                rows_v.at[b, :, pl.ds(0, DO)],
                out_hbm.at[base + i],
                ssem[b],
            )

        def wait_store(b):
            pltpu.make_async_copy(
                rows_v.at[b, :, pl.ds(0, DO)],
                out_hbm.at[base],
                ssem[b],
            ).wait()

        start_gather(0, 0)
        start_gather(1, 1)

        def body(j, carry):
            for b in range(2):
                i = j * 2 + b
                wait_gather(b)
                start_store(i, b)

                @pl.when(i + 2 < bpw)
                def _():
                    wait_store(b)
                    start_gather(i + 2, b)

            return carry

        lax.fori_loop(0, bpw // 2, body, 0)
        wait_store(0)
        wait_store(1)

    return k


def kernel(idx, table):
    batch, seq = idx.shape
    v, d = table.shape
    sp = -(-seq // 8) * 8
    do = -(-d // 128) * 128
    db = do // 128 + 1
    idx_p = jnp.pad(idx, ((0, 0), (0, sp - seq)))
    tab3 = jnp.pad(table, ((0, 0), (0, db * 128 - d))).reshape(v, db, 128)
    out = _build_gather(batch, sp, v, db)(idx_p, tab3)
    y = out.reshape(batch, sp, do)
    return y[:, :seq, :d]


# speedup vs baseline: 1.8914x; 1.8914x over previous
import functools

import jax
import jax.numpy as jnp
from jax import lax
from jax.experimental import pallas as pl
from jax.experimental.pallas import tpu as pltpu
from jax.experimental.pallas import tpu_sc as plsc


@functools.lru_cache(maxsize=None)
def _build_gather(BATCH: int, SEQ: int, SP: int, V: int, DB: int, DW: int):
    info = plsc.get_sparse_core_info()
    nc, ns = info.num_cores, info.num_subcores
    nw = nc * ns
    assert BATCH % nw == 0
    bpw = BATCH // nw
    assert bpw % 2 == 0 and bpw >= 4
    DO = DB - 1

    mesh = plsc.VectorSubcoreMesh(core_axis_name="c", subcore_axis_name="s")

    @functools.partial(
        pl.kernel,
        mesh=mesh,
        compiler_params=pltpu.CompilerParams(use_tc_tiling_on_sc=False),
        out_type=jax.ShapeDtypeStruct((BATCH, SP, DO, 128), jnp.float32),
        scratch_types=[
            pltpu.VMEM((bpw, SEQ), jnp.int32),
            pltpu.VMEM((2, SEQ, DB, DW), jnp.float32),
            pltpu.SemaphoreType.DMA,
            pltpu.SemaphoreType.DMA,
            pltpu.SemaphoreType.DMA,
            pltpu.SemaphoreType.DMA,
        ],
    )
    def k(idx_hbm, table_hbm, out_hbm, idx_v, rows_v, g0, g1, s0, s1):
        gsem = (g0, g1)
        ssem = (s0, s1)
        wid = lax.axis_index("s") * nc + lax.axis_index("c")
        base = wid * bpw
        pltpu.sync_copy(idx_hbm.at[pl.ds(base, bpw)], idx_v)

        def start_gather(i, b):
            pltpu.async_copy(table_hbm.at[idx_v.at[i]], rows_v.at[b], gsem[b])

        def wait_gather(b):
            pltpu.make_async_copy(
                table_hbm.at[pl.ds(0, SEQ)], rows_v.at[b], gsem[b]
            ).wait()

        def start_store(i, b):
            pltpu.async_copy(
                rows_v.at[b, :, pl.ds(0, DO), pl.ds(0, 128)],
                out_hbm.at[base + i, pl.ds(0, SEQ)],
                ssem[b],
            )

        def wait_store(b):
            pltpu.make_async_copy(
                rows_v.at[b, :, pl.ds(0, DO), pl.ds(0, 128)],
                out_hbm.at[base, pl.ds(0, SEQ)],
                ssem[b],
            ).wait()

        start_gather(0, 0)
        start_gather(1, 1)

        def body(j, carry):
            for b in range(2):
                i = j * 2 + b
                wait_gather(b)
                start_store(i, b)

                @pl.when(i + 2 < bpw)
                def _():
                    wait_store(b)
                    start_gather(i + 2, b)

            return carry

        lax.fori_loop(0, bpw // 2, body, 0)
        wait_store(0)
        wait_store(1)

    return k


def kernel(idx, table):
    batch, seq = idx.shape
    v, d = table.shape
    sp = -(-seq // 8) * 8
    do = -(-d // 128) * 128
    db = do // 128 + 1
    dw = 136
    t = jnp.pad(table, ((0, 0), (0, do - d))).reshape(v, do // 128, 128)
    tab3 = jnp.pad(t, ((0, 0), (0, 1), (0, dw - 128)))
    out = _build_gather(batch, seq, sp, v, db, dw)(idx, tab3)
    y = out.reshape(batch, sp, do)
    return y[:, :seq, :d]


# speedup vs baseline: 1.9641x; 1.0384x over previous
import functools

import jax
import jax.numpy as jnp
from jax import lax
from jax.experimental import pallas as pl
from jax.experimental.pallas import tpu as pltpu
from jax.experimental.pallas import tpu_sc as plsc


@functools.lru_cache(maxsize=None)
def _build_gather(BATCH: int, SEQ: int, V: int, DB: int, DW: int):
    info = plsc.get_sparse_core_info()
    nc, ns = info.num_cores, info.num_subcores
    nw = nc * ns
    assert BATCH % nw == 0
    bpw = BATCH // nw
    assert bpw % 2 == 0 and bpw >= 4
    DO = DB - 1

    mesh = plsc.VectorSubcoreMesh(core_axis_name="c", subcore_axis_name="s")

    @functools.partial(
        pl.kernel,
        mesh=mesh,
        compiler_params=pltpu.CompilerParams(use_tc_tiling_on_sc=False),
        out_type=jax.ShapeDtypeStruct((BATCH, SEQ, DO, 128), jnp.float32),
        scratch_types=[
            pltpu.VMEM((bpw, SEQ), jnp.int32),
            pltpu.VMEM((2, SEQ, DB, DW), jnp.float32),
            pltpu.SemaphoreType.DMA,
            pltpu.SemaphoreType.DMA,
            pltpu.SemaphoreType.DMA,
            pltpu.SemaphoreType.DMA,
        ],
    )
    def k(idx_hbm, table_hbm, out_hbm, idx_v, rows_v, g0, g1, s0, s1):
        gsem = (g0, g1)
        ssem = (s0, s1)
        wid = lax.axis_index("s") * nc + lax.axis_index("c")
        base = wid * bpw
        pltpu.sync_copy(idx_hbm.at[pl.ds(base, bpw)], idx_v)

        def start_gather(i, b):
            pltpu.async_copy(table_hbm.at[idx_v.at[i]], rows_v.at[b], gsem[b])

        def wait_gather(b):
            pltpu.make_async_copy(
                table_hbm.at[pl.ds(0, SEQ)], rows_v.at[b], gsem[b]
            ).wait()

        def start_store(i, b):
            pltpu.async_copy(
                rows_v.at[b, :, pl.ds(0, DO), pl.ds(0, 128)],
                out_hbm.at[base + i],
                ssem[b],
            )

        def wait_store(b):
            pltpu.make_async_copy(
                rows_v.at[b, :, pl.ds(0, DO), pl.ds(0, 128)],
                out_hbm.at[base],
                ssem[b],
            ).wait()

        start_gather(0, 0)
        start_gather(1, 1)

        def body(j, carry):
            for b in range(2):
                i = j * 2 + b
                wait_gather(b)
                start_store(i, b)

                @pl.when(i + 2 < bpw)
                def _():
                    wait_store(b)
                    start_gather(i + 2, b)

            return carry

        lax.fori_loop(0, bpw // 2, body, 0)
        wait_store(0)
        wait_store(1)

    return k


def kernel(idx, table):
    batch, seq = idx.shape
    v, d = table.shape
    do = -(-d // 128) * 128
    db = do // 128 + 1
    dw = 136
    t = jnp.pad(table, ((0, 0), (0, do - d))).reshape(v, do // 128, 128)
    tab3 = jnp.pad(t, ((0, 0), (0, 1), (0, dw - 128)))
    out = _build_gather(batch, seq, v, db, dw)(idx, tab3)
    y = out.reshape(batch, seq, do)
    return y[:, :, :d]
